# Initial kernel scaffold; baseline (speedup 1.0000x reference)
import jax
import jax.numpy as jnp
from jax.experimental import pallas as pl


def kernel(inputs, indices, W, gamma, beta):
    raise NotImplementedError("write your pallas kernel here")


# speedup vs baseline: 5.7079x; 5.7079x over previous
import jax
import jax.numpy as jnp
from jax.experimental import pallas as pl
from jax.experimental.pallas import tpu as pltpu

_EPS = 1e-3


def _body(xin_ref, idx_ref, w_ref, g_ref, b_ref, o_ref):
    bs = o_ref.shape[0]
    n = o_ref.shape[1]
    u = w_ref.shape[0]

    xin = xin_ref[...]
    w = w_ref[...]
    xt = jax.lax.dot_general(
        w, xin,
        dimension_numbers=(((1,), (1,)), ((), ())),
        preferred_element_type=jnp.float32,
    )
    mean = jnp.mean(xt, axis=1, keepdims=True)
    var = jnp.mean(xt * xt, axis=1, keepdims=True) - mean * mean
    xt = (xt - mean) * jax.lax.rsqrt(var + _EPS) * g_ref[...] + b_ref[...]
    xt = jnp.maximum(xt, 0.0)

    idx = idx_ref[...]

    for b in range(bs):
        xb = jax.lax.slice(xt, (0, b * n), (u, (b + 1) * n))
        ib = jax.lax.slice(idx, (0, b * n), (1, (b + 1) * n))

        f = xb
        d = 1
        while d < n:
            ish = jnp.concatenate(
                [jnp.full((1, d), -1, jnp.int32),
                 jax.lax.slice(ib, (0, 0), (1, n - d))], axis=1)
            same = ib == ish
            sh = jnp.concatenate(
                [jnp.zeros((u, d), jnp.float32),
                 jax.lax.slice(f, (0, 0), (u, n - d))], axis=1)
            f = jnp.maximum(f, jnp.where(same, sh, 0.0))
            d *= 2

        r = f
        d = 1
        while d < n:
            ish = jnp.concatenate(
                [jax.lax.slice(ib, (0, d), (1, n)),
                 jnp.full((1, d), -1, jnp.int32)], axis=1)
            same = ib == ish
            sh = jnp.concatenate(
                [jax.lax.slice(r, (0, d), (u, n)),
                 jnp.zeros((u, d), jnp.float32)], axis=1)
            r = jnp.maximum(r, jnp.where(same, sh, 0.0))
            d *= 2

        ob = jnp.concatenate([xb, r], axis=0)
        o_ref[b] = jnp.transpose(ob)


def kernel(inputs, indices, W, gamma, beta):
    bs, n, cin = inputs.shape
    u = W.shape[0]
    xin2d = inputs.reshape(bs * n, cin)
    idx2d = indices.astype(jnp.int32).reshape(1, bs * n)
    g = gamma.reshape(u, 1)
    b = beta.reshape(u, 1)
    out = pl.pallas_call(
        _body,
        out_shape=jax.ShapeDtypeStruct((bs, n, 2 * u), jnp.float32),
        compiler_params=pltpu.CompilerParams(
            vmem_limit_bytes=100 * 1024 * 1024),
    )(xin2d, idx2d, W, g, b)
    return out
